# initial kernel scaffold (unmeasured)
import jax
import jax.numpy as jnp
from jax import lax
from jax.experimental import pallas as pl
from jax.experimental.pallas import tpu as pltpu

N_X = 2
T = 512
D = 512
F = 1024
E_LOCAL = 2


def kernel(x, assign, W1, W2):
    assign2d = assign.reshape(T, 1)

    def body(x_ref, a_ref, w1_ref, w2_ref, out_ref,
             xg_ref, ag_ref, ob_ref, send_sems, recv_sems):
        my_x = lax.axis_index("x")
        my_y = lax.axis_index("y")
        peer_x = 1 - my_x
        my_off = my_x * T
        peer_off = peer_x * T

        barrier_sem = pltpu.get_barrier_semaphore()
        pl.semaphore_signal(
            barrier_sem, inc=1,
            device_id=(peer_x, my_y), device_id_type=pl.DeviceIdType.MESH,
        )
        pl.semaphore_wait(barrier_sem, 1)

        xg_ref[pl.ds(my_off, T), :] = x_ref[:, :].astype(jnp.bfloat16)
        ag_ref[pl.ds(my_off, T), :] = a_ref[:, :]

        rdma_x = pltpu.make_async_remote_copy(
            src_ref=xg_ref.at[pl.ds(my_off, T), :],
            dst_ref=xg_ref.at[pl.ds(my_off, T), :],
            send_sem=send_sems.at[0],
            recv_sem=recv_sems.at[0],
            device_id=(peer_x, my_y),
            device_id_type=pl.DeviceIdType.MESH,
        )
        rdma_a = pltpu.make_async_remote_copy(
            src_ref=ag_ref.at[pl.ds(my_off, T), :],
            dst_ref=ag_ref.at[pl.ds(my_off, T), :],
            send_sem=send_sems.at[1],
            recv_sem=recv_sems.at[1],
            device_id=(peer_x, my_y),
            device_id_type=pl.DeviceIdType.MESH,
        )
        rdma_x.start()
        rdma_a.start()
        rdma_x.wait()
        rdma_a.wait()

        xg = xg_ref[:, :]
        ag = ag_ref[:, :]
        acc = jnp.zeros((N_X * T, D), jnp.float32)
        for k in range(E_LOCAL):
            e = my_x * E_LOCAL + k
            mask = (ag == e).astype(jnp.bfloat16)
            xm = xg * mask
            h = jnp.dot(xm, w1_ref[k].astype(jnp.bfloat16),
                        preferred_element_type=jnp.float32)
            hb = jnp.maximum(h, 0.0).astype(jnp.bfloat16)
            acc = acc + jnp.dot(hb, w2_ref[k].astype(jnp.bfloat16),
                                preferred_element_type=jnp.float32)

        ob_ref[0] = lax.dynamic_slice(acc, (peer_off, 0), (T, D)).astype(
            jnp.bfloat16)
        rdma_o = pltpu.make_async_remote_copy(
            src_ref=ob_ref.at[0],
            dst_ref=ob_ref.at[1],
            send_sem=send_sems.at[2],
            recv_sem=recv_sems.at[2],
            device_id=(peer_x, my_y),
            device_id_type=pl.DeviceIdType.MESH,
        )
        rdma_o.start()
        rdma_o.wait()

        out_ref[:, :] = (
            lax.dynamic_slice(acc, (my_off, 0), (T, D))
            + ob_ref[1].astype(jnp.float32)
        )

    return pl.pallas_call(
        body,
        out_shape=jax.ShapeDtypeStruct((T, D), jnp.float32),
        in_specs=[pl.BlockSpec(memory_space=pltpu.VMEM)] * 4,
        out_specs=pl.BlockSpec(memory_space=pltpu.VMEM),
        scratch_shapes=[
            pltpu.VMEM((N_X * T, D), jnp.bfloat16),
            pltpu.VMEM((N_X * T, 1), jnp.int32),
            pltpu.VMEM((2, T, D), jnp.bfloat16),
            pltpu.SemaphoreType.DMA((3,)),
            pltpu.SemaphoreType.DMA((3,)),
        ],
        compiler_params=pltpu.CompilerParams(collective_id=0),
    )(x, assign2d, W1, W2)


# baseline (device time: 32725 ns/iter reference)
import jax
import jax.numpy as jnp
from jax import lax
from jax.experimental import pallas as pl
from jax.experimental.pallas import tpu as pltpu

N_X = 2
T = 512
D = 512
F = 1024
E_LOCAL = 2


def kernel(x, assign, W1, W2):
    assign2d = assign.reshape(T, 1)

    def body(x_ref, a_ref, w1_ref, w2_ref, out_ref,
             xg_ref, ag_ref, ob_ref, acc_ref, send_sems, recv_sems):
        my_x = lax.axis_index("x")
        my_y = lax.axis_index("y")
        peer_x = 1 - my_x
        my_off = my_x * T
        peer_off = peer_x * T

        barrier_sem = pltpu.get_barrier_semaphore()
        pl.semaphore_signal(
            barrier_sem, inc=1,
            device_id=(peer_x, my_y), device_id_type=pl.DeviceIdType.MESH,
        )
        pl.semaphore_wait(barrier_sem, 1)

        xg_ref[pl.ds(my_off, T), :] = x_ref[:, :].astype(jnp.bfloat16)
        ag_ref[pl.ds(my_off, T), :] = a_ref[:, :]

        rdma_x = pltpu.make_async_remote_copy(
            src_ref=xg_ref.at[pl.ds(my_off, T), :],
            dst_ref=xg_ref.at[pl.ds(my_off, T), :],
            send_sem=send_sems.at[0],
            recv_sem=recv_sems.at[0],
            device_id=(peer_x, my_y),
            device_id_type=pl.DeviceIdType.MESH,
        )
        rdma_a = pltpu.make_async_remote_copy(
            src_ref=ag_ref.at[pl.ds(my_off, T), :],
            dst_ref=ag_ref.at[pl.ds(my_off, T), :],
            send_sem=send_sems.at[1],
            recv_sem=recv_sems.at[1],
            device_id=(peer_x, my_y),
            device_id_type=pl.DeviceIdType.MESH,
        )
        rdma_x.start()
        rdma_a.start()
        rdma_x.wait()
        rdma_a.wait()

        xg = xg_ref[:, :]
        ag = ag_ref[:, :]
        acc = jnp.zeros((N_X * T, D), jnp.float32)
        for k in range(E_LOCAL):
            e = my_x * E_LOCAL + k
            mask = (ag == e).astype(jnp.bfloat16)
            xm = xg * mask
            h = jnp.dot(xm, w1_ref[k].astype(jnp.bfloat16),
                        preferred_element_type=jnp.float32)
            hb = jnp.maximum(h, 0.0).astype(jnp.bfloat16)
            acc = acc + jnp.dot(hb, w2_ref[k].astype(jnp.bfloat16),
                                preferred_element_type=jnp.float32)
        acc_ref[:, :] = acc

        ob_ref[0] = acc_ref[pl.ds(peer_off, T), :].astype(jnp.bfloat16)
        rdma_o = pltpu.make_async_remote_copy(
            src_ref=ob_ref.at[0],
            dst_ref=ob_ref.at[1],
            send_sem=send_sems.at[2],
            recv_sem=recv_sems.at[2],
            device_id=(peer_x, my_y),
            device_id_type=pl.DeviceIdType.MESH,
        )
        rdma_o.start()
        rdma_o.wait()

        out_ref[:, :] = (
            acc_ref[pl.ds(my_off, T), :] + ob_ref[1].astype(jnp.float32)
        )

    return pl.pallas_call(
        body,
        out_shape=jax.ShapeDtypeStruct((T, D), jnp.float32),
        in_specs=[pl.BlockSpec(memory_space=pltpu.VMEM)] * 4,
        out_specs=pl.BlockSpec(memory_space=pltpu.VMEM),
        scratch_shapes=[
            pltpu.VMEM((N_X * T, D), jnp.bfloat16),
            pltpu.VMEM((N_X * T, 1), jnp.int32),
            pltpu.VMEM((2, T, D), jnp.bfloat16),
            pltpu.VMEM((N_X * T, D), jnp.float32),
            pltpu.SemaphoreType.DMA((3,)),
            pltpu.SemaphoreType.DMA((3,)),
        ],
        compiler_params=pltpu.CompilerParams(collective_id=0),
    )(x, assign2d, W1, W2)


# device time: 29234 ns/iter; 1.1194x vs baseline; 1.1194x over previous
import jax
import jax.numpy as jnp
from jax import lax
from jax.experimental import pallas as pl
from jax.experimental.pallas import tpu as pltpu

N_X = 2
T = 512
D = 512
F = 1024
E_LOCAL = 2
CHUNKS = 2
TC = T // CHUNKS


def kernel(x, assign, W1, W2):
    assign2d = assign.reshape(T, 1)

    def body(x_ref, a_ref, w1_ref, w2_ref, out_ref,
             xg_ref, ag_ref, ob_ref, send_sems, recv_sems):
        my_x = lax.axis_index("x")
        my_y = lax.axis_index("y")
        peer_x = 1 - my_x
        my_off = my_x * T
        peer_off = peer_x * T

        barrier_sem = pltpu.get_barrier_semaphore()
        pl.semaphore_signal(
            barrier_sem, inc=1,
            device_id=(peer_x, my_y), device_id_type=pl.DeviceIdType.MESH,
        )
        pl.semaphore_wait(barrier_sem, 1)

        xg_ref[pl.ds(my_off, T), :] = x_ref[:, :].astype(jnp.bfloat16)
        ag_ref[pl.ds(my_off, T), :] = a_ref[:, :]
        rdma_x = pltpu.make_async_remote_copy(
            src_ref=xg_ref.at[pl.ds(my_off, T), :],
            dst_ref=xg_ref.at[pl.ds(my_off, T), :],
            send_sem=send_sems.at[0],
            recv_sem=recv_sems.at[0],
            device_id=(peer_x, my_y),
            device_id_type=pl.DeviceIdType.MESH,
        )
        rdma_a = pltpu.make_async_remote_copy(
            src_ref=ag_ref.at[pl.ds(my_off, T), :],
            dst_ref=ag_ref.at[pl.ds(my_off, T), :],
            send_sem=send_sems.at[1],
            recv_sem=recv_sems.at[1],
            device_id=(peer_x, my_y),
            device_id_type=pl.DeviceIdType.MESH,
        )
        rdma_x.start()
        rdma_a.start()

        w1 = [w1_ref[k].astype(jnp.bfloat16) for k in range(E_LOCAL)]
        w2 = [w2_ref[k].astype(jnp.bfloat16) for k in range(E_LOCAL)]

        def contrib(xrows, arows):
            acc = jnp.zeros((xrows.shape[0], D), jnp.float32)
            for k in range(E_LOCAL):
                e = my_x * E_LOCAL + k
                xm = xrows * (arows == e).astype(jnp.bfloat16)
                h = jnp.dot(xm, w1[k], preferred_element_type=jnp.float32)
                hb = jnp.maximum(h, 0.0).astype(jnp.bfloat16)
                acc = acc + jnp.dot(hb, w2[k],
                                    preferred_element_type=jnp.float32)
            return acc

        mine = contrib(x_ref[:, :].astype(jnp.bfloat16), a_ref[:, :])

        rdma_x.wait()
        rdma_a.wait()

        out_rdmas = []
        for c in range(CHUNKS):
            off = peer_off + c * TC
            part = contrib(xg_ref[pl.ds(off, TC), :],
                           ag_ref[pl.ds(off, TC), :])
            ob_ref[0, pl.ds(c * TC, TC), :] = part.astype(jnp.bfloat16)
            rdma_o = pltpu.make_async_remote_copy(
                src_ref=ob_ref.at[0, pl.ds(c * TC, TC), :],
                dst_ref=ob_ref.at[1, pl.ds(c * TC, TC), :],
                send_sem=send_sems.at[2 + c],
                recv_sem=recv_sems.at[2 + c],
                device_id=(peer_x, my_y),
                device_id_type=pl.DeviceIdType.MESH,
            )
            rdma_o.start()
            out_rdmas.append(rdma_o)
        for rdma_o in out_rdmas:
            rdma_o.wait()

        out_ref[:, :] = mine + ob_ref[1].astype(jnp.float32)

    return pl.pallas_call(
        body,
        out_shape=jax.ShapeDtypeStruct((T, D), jnp.float32),
        in_specs=[pl.BlockSpec(memory_space=pltpu.VMEM)] * 4,
        out_specs=pl.BlockSpec(memory_space=pltpu.VMEM),
        scratch_shapes=[
            pltpu.VMEM((N_X * T, D), jnp.bfloat16),
            pltpu.VMEM((N_X * T, 1), jnp.int32),
            pltpu.VMEM((2, T, D), jnp.bfloat16),
            pltpu.SemaphoreType.DMA((2 + CHUNKS,)),
            pltpu.SemaphoreType.DMA((2 + CHUNKS,)),
        ],
        compiler_params=pltpu.CompilerParams(collective_id=0),
    )(x, assign2d, W1, W2)


# device time: 27365 ns/iter; 1.1959x vs baseline; 1.0683x over previous
import jax
import jax.numpy as jnp
from jax import lax
from jax.experimental import pallas as pl
from jax.experimental.pallas import tpu as pltpu

N_X = 2
T = 512
D = 512
F = 1024
E_LOCAL = 2
CHUNKS = 4
TC = T // CHUNKS


def kernel(x, assign, W1, W2):
    assign2d = assign.astype(jnp.bfloat16).reshape(T, 1)

    def body(x_ref, a_ref, w1_ref, w2_ref, out_ref,
             xg_ref, ag_ref, ob_ref, send_sems, recv_sems):
        my_x = lax.axis_index("x")
        my_y = lax.axis_index("y")
        peer_x = 1 - my_x
        my_off = my_x * T
        peer_off = peer_x * T

        barrier_sem = pltpu.get_barrier_semaphore()
        pl.semaphore_signal(
            barrier_sem, inc=1,
            device_id=(peer_x, my_y), device_id_type=pl.DeviceIdType.MESH,
        )
        pl.semaphore_wait(barrier_sem, 1)

        xg_ref[pl.ds(my_off, T), :] = x_ref[:, :].astype(jnp.bfloat16)
        ag_ref[pl.ds(my_off, T), :] = a_ref[:, :]
        rdma_x = pltpu.make_async_remote_copy(
            src_ref=xg_ref.at[pl.ds(my_off, T), :],
            dst_ref=xg_ref.at[pl.ds(my_off, T), :],
            send_sem=send_sems.at[0],
            recv_sem=recv_sems.at[0],
            device_id=(peer_x, my_y),
            device_id_type=pl.DeviceIdType.MESH,
        )
        rdma_a = pltpu.make_async_remote_copy(
            src_ref=ag_ref.at[pl.ds(my_off, T), :],
            dst_ref=ag_ref.at[pl.ds(my_off, T), :],
            send_sem=send_sems.at[1],
            recv_sem=recv_sems.at[1],
            device_id=(peer_x, my_y),
            device_id_type=pl.DeviceIdType.MESH,
        )
        rdma_x.start()
        rdma_a.start()

        w1 = [w1_ref[k].astype(jnp.bfloat16) for k in range(E_LOCAL)]
        w2 = [w2_ref[k].astype(jnp.bfloat16) for k in range(E_LOCAL)]

        def contrib(xrows, arows):
            acc = jnp.zeros((xrows.shape[0], D), jnp.float32)
            for k in range(E_LOCAL):
                e = my_x * E_LOCAL + k
                xm = xrows * (arows == e.astype(jnp.bfloat16)).astype(
                    jnp.bfloat16)
                h = jnp.dot(xm, w1[k], preferred_element_type=jnp.float32)
                hb = jnp.maximum(h, 0.0).astype(jnp.bfloat16)
                acc = acc + jnp.dot(hb, w2[k],
                                    preferred_element_type=jnp.float32)
            return acc

        mine = contrib(x_ref[:, :].astype(jnp.bfloat16), a_ref[:, :])

        rdma_x.wait()
        rdma_a.wait()

        a_peer = ag_ref[pl.ds(peer_off, T), :]

        out_rdmas = []
        for c in range(CHUNKS):
            part = contrib(xg_ref[pl.ds(peer_off + c * TC, TC), :],
                           a_peer[c * TC:(c + 1) * TC, :])
            ob_ref[0, pl.ds(c * TC, TC), :] = part.astype(jnp.bfloat16)
            rdma_o = pltpu.make_async_remote_copy(
                src_ref=ob_ref.at[0, pl.ds(c * TC, TC), :],
                dst_ref=ob_ref.at[1, pl.ds(c * TC, TC), :],
                send_sem=send_sems.at[2 + c],
                recv_sem=recv_sems.at[2 + c],
                device_id=(peer_x, my_y),
                device_id_type=pl.DeviceIdType.MESH,
            )
            rdma_o.start()
            out_rdmas.append(rdma_o)
        for rdma_o in out_rdmas:
            rdma_o.wait()

        out_ref[:, :] = mine + ob_ref[1].astype(jnp.float32)

    return pl.pallas_call(
        body,
        out_shape=jax.ShapeDtypeStruct((T, D), jnp.float32),
        in_specs=[pl.BlockSpec(memory_space=pltpu.VMEM)] * 4,
        out_specs=pl.BlockSpec(memory_space=pltpu.VMEM),
        scratch_shapes=[
            pltpu.VMEM((N_X * T, D), jnp.bfloat16),
            pltpu.VMEM((N_X * T, 1), jnp.bfloat16),
            pltpu.VMEM((2, T, D), jnp.bfloat16),
            pltpu.SemaphoreType.DMA((2 + CHUNKS,)),
            pltpu.SemaphoreType.DMA((2 + CHUNKS,)),
        ],
        compiler_params=pltpu.CompilerParams(collective_id=0),
    )(x, assign2d, W1, W2)


# device time: 25158 ns/iter; 1.3008x vs baseline; 1.0877x over previous
import jax
import jax.numpy as jnp
from jax import lax
from jax.experimental import pallas as pl
from jax.experimental.pallas import tpu as pltpu

N_X = 2
T = 512
D = 512
F = 1024
E_LOCAL = 2
XCHUNKS = 2
XC = T // XCHUNKS
CHUNKS = 4
TC = T // CHUNKS


def kernel(x, assign, W1, W2):
    assign2d = assign.astype(jnp.bfloat16).reshape(T, 1)

    def body(x_ref, a_ref, w1_hbm, w2_hbm, out_ref,
             xg_ref, ag_ref, ob_ref, w1_ref, w2_ref,
             send_sems, recv_sems, w_sems):
        my_x = lax.axis_index("x")
        my_y = lax.axis_index("y")
        peer_x = 1 - my_x
        my_off = my_x * T
        peer_off = peer_x * T

        w1_dma = pltpu.make_async_copy(w1_hbm, w1_ref, w_sems.at[0])
        w2_dma = pltpu.make_async_copy(w2_hbm, w2_ref, w_sems.at[1])
        w1_dma.start()
        w2_dma.start()

        barrier_sem = pltpu.get_barrier_semaphore()
        pl.semaphore_signal(
            barrier_sem, inc=1,
            device_id=(peer_x, my_y), device_id_type=pl.DeviceIdType.MESH,
        )
        pl.semaphore_wait(barrier_sem, 1)

        xg_ref[pl.ds(my_off, T), :] = x_ref[:, :].astype(jnp.bfloat16)
        ag_ref[pl.ds(my_off, T), :] = a_ref[:, :]
        rdma_a = pltpu.make_async_remote_copy(
            src_ref=ag_ref.at[pl.ds(my_off, T), :],
            dst_ref=ag_ref.at[pl.ds(my_off, T), :],
            send_sem=send_sems.at[0],
            recv_sem=recv_sems.at[0],
            device_id=(peer_x, my_y),
            device_id_type=pl.DeviceIdType.MESH,
        )
        rdma_a.start()
        rdma_x = []
        for c in range(XCHUNKS):
            r = pltpu.make_async_remote_copy(
                src_ref=xg_ref.at[pl.ds(my_off + c * XC, XC), :],
                dst_ref=xg_ref.at[pl.ds(my_off + c * XC, XC), :],
                send_sem=send_sems.at[1 + c],
                recv_sem=recv_sems.at[1 + c],
                device_id=(peer_x, my_y),
                device_id_type=pl.DeviceIdType.MESH,
            )
            r.start()
            rdma_x.append(r)

        w1_dma.wait()
        w2_dma.wait()
        w1 = [w1_ref[k].astype(jnp.bfloat16) for k in range(E_LOCAL)]
        w2 = [w2_ref[k].astype(jnp.bfloat16) for k in range(E_LOCAL)]

        def contrib(xrows, arows):
            acc = jnp.zeros((xrows.shape[0], D), jnp.float32)
            for k in range(E_LOCAL):
                e = my_x * E_LOCAL + k
                xm = xrows * (arows == e.astype(jnp.bfloat16)).astype(
                    jnp.bfloat16)
                h = jnp.dot(xm, w1[k], preferred_element_type=jnp.float32)
                hb = jnp.maximum(h, 0.0).astype(jnp.bfloat16)
                acc = acc + jnp.dot(hb, w2[k],
                                    preferred_element_type=jnp.float32)
            return acc

        mine = contrib(x_ref[:, :].astype(jnp.bfloat16), a_ref[:, :])

        rdma_a.wait()

        out_rdmas = []
        for c in range(CHUNKS):
            xc = c * TC // XC
            if c == 0 or xc != ((c - 1) * TC) // XC:
                rdma_x[xc].wait()
            part = contrib(xg_ref[pl.ds(peer_off + c * TC, TC), :],
                           ag_ref[pl.ds(peer_off + c * TC, TC), :])
            ob_ref[0, pl.ds(c * TC, TC), :] = part.astype(jnp.bfloat16)
            rdma_o = pltpu.make_async_remote_copy(
                src_ref=ob_ref.at[0, pl.ds(c * TC, TC), :],
                dst_ref=ob_ref.at[1, pl.ds(c * TC, TC), :],
                send_sem=send_sems.at[1 + XCHUNKS + c],
                recv_sem=recv_sems.at[1 + XCHUNKS + c],
                device_id=(peer_x, my_y),
                device_id_type=pl.DeviceIdType.MESH,
            )
            rdma_o.start()
            out_rdmas.append(rdma_o)
        for rdma_o in out_rdmas:
            rdma_o.wait()

        out_ref[:, :] = mine + ob_ref[1].astype(jnp.float32)

    nsem = 1 + XCHUNKS + CHUNKS
    return pl.pallas_call(
        body,
        out_shape=jax.ShapeDtypeStruct((T, D), jnp.float32),
        in_specs=[
            pl.BlockSpec(memory_space=pltpu.VMEM),
            pl.BlockSpec(memory_space=pltpu.VMEM),
            pl.BlockSpec(memory_space=pl.ANY),
            pl.BlockSpec(memory_space=pl.ANY),
        ],
        out_specs=pl.BlockSpec(memory_space=pltpu.VMEM),
        scratch_shapes=[
            pltpu.VMEM((N_X * T, D), jnp.bfloat16),
            pltpu.VMEM((N_X * T, 1), jnp.bfloat16),
            pltpu.VMEM((2, T, D), jnp.bfloat16),
            pltpu.VMEM((E_LOCAL, D, F), jnp.float32),
            pltpu.VMEM((E_LOCAL, F, D), jnp.float32),
            pltpu.SemaphoreType.DMA((nsem,)),
            pltpu.SemaphoreType.DMA((nsem,)),
            pltpu.SemaphoreType.DMA((2,)),
        ],
        compiler_params=pltpu.CompilerParams(collective_id=0),
    )(x, assign2d, W1, W2)


# device time: 19688 ns/iter; 1.6622x vs baseline; 1.2778x over previous
import jax
import jax.numpy as jnp
from jax import lax
from jax.experimental import pallas as pl
from jax.experimental.pallas import tpu as pltpu

N_X = 2
T = 512
D = 512
F = 1024
E_LOCAL = 2
XCHUNKS = 2
XC = T // XCHUNKS
CHUNKS = 4
TC = T // CHUNKS


def kernel(x, assign, W1, W2):
    assign2d = assign.reshape(T, 1)
    x = pltpu.with_memory_space_constraint(x, pltpu.MemorySpace.HBM)
    W1 = pltpu.with_memory_space_constraint(W1, pltpu.MemorySpace.HBM)
    W2 = pltpu.with_memory_space_constraint(W2, pltpu.MemorySpace.HBM)

    def body(x_hbm, a_ref, w1_hbm, w2_hbm, out_ref,
             xg_ref, ag_ref, ob_ref, x_ref, w1_ref, w2_ref,
             send_sems, recv_sems, w_sems):
        my_x = lax.axis_index("x")
        my_y = lax.axis_index("y")
        peer_x = 1 - my_x
        my_off = my_x * T
        peer_off = peer_x * T

        x_dma = pltpu.make_async_copy(x_hbm, x_ref, w_sems.at[2])
        w1_dma = pltpu.make_async_copy(w1_hbm, w1_ref, w_sems.at[0])
        w2_dma = pltpu.make_async_copy(w2_hbm, w2_ref, w_sems.at[1])
        x_dma.start()
        w1_dma.start()
        w2_dma.start()

        barrier_sem = pltpu.get_barrier_semaphore()
        pl.semaphore_signal(
            barrier_sem, inc=1,
            device_id=(peer_x, my_y), device_id_type=pl.DeviceIdType.MESH,
        )
        pl.semaphore_wait(barrier_sem, 1)

        x_dma.wait()
        xg_ref[pl.ds(my_off, T), :] = x_ref[:, :].astype(jnp.bfloat16)
        ag_ref[pl.ds(my_off, T), :] = a_ref[:, :].astype(jnp.bfloat16)
        rdma_a = pltpu.make_async_remote_copy(
            src_ref=ag_ref.at[pl.ds(my_off, T), :],
            dst_ref=ag_ref.at[pl.ds(my_off, T), :],
            send_sem=send_sems.at[0],
            recv_sem=recv_sems.at[0],
            device_id=(peer_x, my_y),
            device_id_type=pl.DeviceIdType.MESH,
        )
        rdma_a.start()
        rdma_x = []
        for c in range(XCHUNKS):
            r = pltpu.make_async_remote_copy(
                src_ref=xg_ref.at[pl.ds(my_off + c * XC, XC), :],
                dst_ref=xg_ref.at[pl.ds(my_off + c * XC, XC), :],
                send_sem=send_sems.at[1 + c],
                recv_sem=recv_sems.at[1 + c],
                device_id=(peer_x, my_y),
                device_id_type=pl.DeviceIdType.MESH,
            )
            r.start()
            rdma_x.append(r)

        w1_dma.wait()
        w2_dma.wait()
        w1 = [w1_ref[k].astype(jnp.bfloat16) for k in range(E_LOCAL)]
        w2 = [w2_ref[k].astype(jnp.bfloat16) for k in range(E_LOCAL)]

        def contrib(xrows, arows):
            acc = jnp.zeros((xrows.shape[0], D), jnp.float32)
            for k in range(E_LOCAL):
                e = my_x * E_LOCAL + k
                xm = xrows * (arows == e).astype(jnp.bfloat16)
                h = jnp.dot(xm, w1[k], preferred_element_type=jnp.float32)
                hb = jnp.maximum(h, 0.0).astype(jnp.bfloat16)
                acc = acc + jnp.dot(hb, w2[k],
                                    preferred_element_type=jnp.float32)
            return acc

        mine = contrib(x_ref[:, :].astype(jnp.bfloat16), a_ref[:, :])


        rdma_a.wait()

        out_rdmas = []
        for c in range(CHUNKS):
            xc = c * TC // XC
            if c == 0 or xc != ((c - 1) * TC) // XC:
                rdma_x[xc].wait()
            part = contrib(xg_ref[pl.ds(peer_off + c * TC, TC), :],
                           ag_ref[pl.ds(peer_off + c * TC, TC), :])
            ob_ref[0, pl.ds(c * TC, TC), :] = part.astype(jnp.bfloat16)
            rdma_o = pltpu.make_async_remote_copy(
                src_ref=ob_ref.at[0, pl.ds(c * TC, TC), :],
                dst_ref=ob_ref.at[1, pl.ds(c * TC, TC), :],
                send_sem=send_sems.at[1 + XCHUNKS + c],
                recv_sem=recv_sems.at[1 + XCHUNKS + c],
                device_id=(peer_x, my_y),
                device_id_type=pl.DeviceIdType.MESH,
            )
            rdma_o.start()
            out_rdmas.append(rdma_o)
        for rdma_o in out_rdmas:
            rdma_o.wait()

        out_ref[:, :] = mine + ob_ref[1].astype(jnp.float32)

    nsem = 1 + XCHUNKS + CHUNKS
    return pl.pallas_call(
        body,
        out_shape=jax.ShapeDtypeStruct((T, D), jnp.float32),
        in_specs=[
            pl.BlockSpec(memory_space=pl.ANY),
            pl.BlockSpec(memory_space=pltpu.VMEM),
            pl.BlockSpec(memory_space=pl.ANY),
            pl.BlockSpec(memory_space=pl.ANY),
        ],
        out_specs=pl.BlockSpec(memory_space=pltpu.VMEM),
        scratch_shapes=[
            pltpu.VMEM((N_X * T, D), jnp.bfloat16),
            pltpu.VMEM((N_X * T, 1), jnp.bfloat16),
            pltpu.VMEM((2, T, D), jnp.bfloat16),
            pltpu.VMEM((T, D), jnp.float32),
            pltpu.VMEM((E_LOCAL, D, F), jnp.float32),
            pltpu.VMEM((E_LOCAL, F, D), jnp.float32),
            pltpu.SemaphoreType.DMA((nsem,)),
            pltpu.SemaphoreType.DMA((nsem,)),
            pltpu.SemaphoreType.DMA((3,)),
        ],
        compiler_params=pltpu.CompilerParams(collective_id=0),
    )(x, assign2d, W1, W2)


# device time: 19513 ns/iter; 1.6771x vs baseline; 1.0090x over previous
import jax
import jax.numpy as jnp
from jax import lax
from jax.experimental import pallas as pl
from jax.experimental.pallas import tpu as pltpu

T = 512
D = 512
F = 1024
E_LOCAL = 2
H = T // 2
RC = 2
HC = H // RC


def kernel(x, assign, W1, W2):
    assign2d = assign.reshape(T, 1)
    x = pltpu.with_memory_space_constraint(x, pltpu.MemorySpace.HBM)
    W1 = pltpu.with_memory_space_constraint(W1, pltpu.MemorySpace.HBM)
    W2 = pltpu.with_memory_space_constraint(W2, pltpu.MemorySpace.HBM)

    def body(x_hbm, a_ref, w1_hbm, w2_hbm, out_ref,
             xs_ref, xp_ref, as_ref, ap_ref, ob_ref, acc_ref,
             x_ref, w1_ref, w2_ref,
             send_sems, recv_sems, ldma_sems):
        my_x = lax.axis_index("x")
        my_y = lax.axis_index("y")
        peer_x = 1 - my_x
        peer_y = 1 - my_y
        yoff = my_y * H

        x_dma = pltpu.make_async_copy(x_hbm, x_ref, ldma_sems.at[2])
        w1_dma = pltpu.make_async_copy(w1_hbm, w1_ref, ldma_sems.at[0])
        w2_dma = pltpu.make_async_copy(w2_hbm, w2_ref, ldma_sems.at[1])
        x_dma.start()
        w1_dma.start()
        w2_dma.start()

        barrier_sem = pltpu.get_barrier_semaphore()
        for tgt in ((peer_x, my_y), (my_x, peer_y), (peer_x, peer_y)):
            pl.semaphore_signal(
                barrier_sem, inc=1,
                device_id=tgt, device_id_type=pl.DeviceIdType.MESH,
            )
        pl.semaphore_wait(barrier_sem, 3)

        x_dma.wait()
        xs_ref[:, :] = x_ref[pl.ds(yoff, H), :].astype(jnp.bfloat16)
        as_ref[:, :] = a_ref[pl.ds(yoff, H), :].astype(jnp.bfloat16)
        rdma_a = pltpu.make_async_remote_copy(
            src_ref=as_ref, dst_ref=ap_ref,
            send_sem=send_sems.at[0], recv_sem=recv_sems.at[0],
            device_id=(peer_x, my_y), device_id_type=pl.DeviceIdType.MESH,
        )
        rdma_a.start()
        rdma_x = []
        for c in range(RC):
            r = pltpu.make_async_remote_copy(
                src_ref=xs_ref.at[pl.ds(c * HC, HC), :],
                dst_ref=xp_ref.at[pl.ds(c * HC, HC), :],
                send_sem=send_sems.at[1 + c],
                recv_sem=recv_sems.at[1 + c],
                device_id=(peer_x, my_y),
                device_id_type=pl.DeviceIdType.MESH,
            )
            r.start()
            rdma_x.append(r)

        w1_dma.wait()
        w2_dma.wait()
        w1 = [w1_ref[k].astype(jnp.bfloat16) for k in range(E_LOCAL)]
        w2 = [w2_ref[k].astype(jnp.bfloat16) for k in range(E_LOCAL)]

        def contrib(xrows, arows):
            acc = jnp.zeros((xrows.shape[0], D), jnp.float32)
            for k in range(E_LOCAL):
                e = my_x * E_LOCAL + k
                xm = xrows * (arows == e).astype(jnp.bfloat16)
                h = jnp.dot(xm, w1[k], preferred_element_type=jnp.float32)
                hb = jnp.maximum(h, 0.0).astype(jnp.bfloat16)
                acc = acc + jnp.dot(hb, w2[k],
                                    preferred_element_type=jnp.float32)
            return acc

        acc_ref[:, :] = contrib(x_ref[:, :].astype(jnp.bfloat16), a_ref[:, :])

        rdma_a.wait()

        out_rdmas = []
        for c in range(RC):
            rdma_x[c].wait()
            part = contrib(xp_ref[pl.ds(c * HC, HC), :],
                           ap_ref[pl.ds(c * HC, HC), :])
            ob_ref[0, pl.ds(c * HC, HC), :] = part.astype(jnp.bfloat16)
            rdma_d = pltpu.make_async_remote_copy(
                src_ref=ob_ref.at[0, pl.ds(c * HC, HC), :],
                dst_ref=ob_ref.at[1, pl.ds(c * HC, HC), :],
                send_sem=send_sems.at[1 + RC + c],
                recv_sem=recv_sems.at[1 + RC + c],
                device_id=(peer_x, my_y),
                device_id_type=pl.DeviceIdType.MESH,
            )
            rdma_g = pltpu.make_async_remote_copy(
                src_ref=ob_ref.at[0, pl.ds(c * HC, HC), :],
                dst_ref=ob_ref.at[2, pl.ds(c * HC, HC), :],
                send_sem=send_sems.at[1 + 2 * RC + c],
                recv_sem=recv_sems.at[1 + 2 * RC + c],
                device_id=(peer_x, peer_y),
                device_id_type=pl.DeviceIdType.MESH,
            )
            rdma_d.start()
            rdma_g.start()
            out_rdmas.extend((rdma_d, rdma_g))
        for r in out_rdmas:
            r.wait()

        out_ref[pl.ds(yoff, H), :] = (
            acc_ref[pl.ds(yoff, H), :] + ob_ref[1].astype(jnp.float32)
        )
        out_ref[pl.ds(peer_y * H, H), :] = (
            acc_ref[pl.ds(peer_y * H, H), :] + ob_ref[2].astype(jnp.float32)
        )

    nsem = 1 + 3 * RC
    return pl.pallas_call(
        body,
        out_shape=jax.ShapeDtypeStruct((T, D), jnp.float32),
        in_specs=[
            pl.BlockSpec(memory_space=pl.ANY),
            pl.BlockSpec(memory_space=pltpu.VMEM),
            pl.BlockSpec(memory_space=pl.ANY),
            pl.BlockSpec(memory_space=pl.ANY),
        ],
        out_specs=pl.BlockSpec(memory_space=pltpu.VMEM),
        scratch_shapes=[
            pltpu.VMEM((H, D), jnp.bfloat16),
            pltpu.VMEM((H, D), jnp.bfloat16),
            pltpu.VMEM((H, 1), jnp.bfloat16),
            pltpu.VMEM((H, 1), jnp.bfloat16),
            pltpu.VMEM((3, H, D), jnp.bfloat16),
            pltpu.VMEM((T, D), jnp.float32),
            pltpu.VMEM((T, D), jnp.float32),
            pltpu.VMEM((E_LOCAL, D, F), jnp.float32),
            pltpu.VMEM((E_LOCAL, F, D), jnp.float32),
            pltpu.SemaphoreType.DMA((nsem,)),
            pltpu.SemaphoreType.DMA((nsem,)),
            pltpu.SemaphoreType.DMA((3,)),
        ],
        compiler_params=pltpu.CompilerParams(collective_id=0),
    )(x, assign2d, W1, W2)
